# baseline (speedup 1.0000x reference)
import jax
import jax.numpy as jnp
import numpy as np
from jax import lax
from jax.experimental import pallas as pl
from jax.experimental.pallas import tpu as pltpu
from jax.experimental.pallas import tpu_sc as plsc

_K = 8192
_D = 256
_NTILE = 16
_CHUNK = _K // _NTILE
_BIG_I = np.int32(2**30)


def _dists_body(z_ref, cb_ref, o_ref):
    d = cb_ref[...] - z_ref[...]
    o_ref[...] = jnp.sum(d * d, axis=1)


def _tc_dists(z_flat, codebook):
    return pl.pallas_call(
        _dists_body,
        grid=(8,),
        in_specs=[
            pl.BlockSpec((1, _D), lambda i: (0, 0)),
            pl.BlockSpec((1024, _D), lambda i: (i, 0)),
        ],
        out_specs=pl.BlockSpec((1024,), lambda i: (i,)),
        out_shape=jax.ShapeDtypeStruct((_K,), jnp.float32),
    )(z_flat.reshape(1, _D), codebook)


def _sc_body(dists_hbm, adj_hbm, sym_hbm, best_hbm, score_hbm,
             dv, lv, li, stage_f, stage_i, sym_v, gidx, ga, gd,
             ob, osc, sh_v, sh_i, sem):
    t = lax.axis_index("s")
    lanes = lax.iota(jnp.int32, 16)
    inf = jnp.float32(jnp.inf)

    pltpu.sync_copy(dists_hbm.at[pl.ds(t * _CHUNK, _CHUNK)], dv)

    loc_v, loc_i = [], []
    for _p in range(4):
        mv = jnp.full((16,), inf, jnp.float32)
        ev = jnp.full((16,), _BIG_I, jnp.int32)
        for j in range(_CHUNK // 16):
            v = dv[pl.ds(j * 16, 16)]
            c = v < mv
            mv = jnp.where(c, v, mv)
            ev = jnp.where(c, lanes + j * 16, ev)
        minval = mv[0]
        minpos = ev[0]
        for l in range(1, 16):
            v = mv[l]
            e = ev[l]
            take = (v < minval) | ((v == minval) & (e < minpos))
            minval = jnp.where(take, v, minval)
            minpos = jnp.where(take, e, minpos)
        loc_v.append(minval)
        loc_i.append(t * _CHUNK + minpos)
        start = (minpos // 16) * 16
        blk = dv[pl.ds(start, 16)]
        dv[pl.ds(start, 16)] = jnp.where(lanes + start == minpos, inf, blk)

    v4 = jnp.full((16,), inf, jnp.float32)
    i4 = jnp.zeros((16,), jnp.int32)
    for p in range(4):
        v4 = jnp.where(lanes == p, loc_v[p], v4)
        i4 = jnp.where(lanes == p, loc_i[p], i4)
    stage_f[...] = v4
    stage_i[...] = i4
    pltpu.sync_copy(stage_f, sh_v.at[pl.ds(t * 16, 16)])
    pltpu.sync_copy(stage_i, sh_i.at[pl.ds(t * 16, 16)])
    plsc.subcore_barrier()

    @pl.when(t == 0)
    def _():
        pltpu.sync_copy(sh_v, lv)
        pltpu.sync_copy(sh_i, li)
        pltpu.sync_copy(sym_hbm, sym_v)

        cand_v, cand_i = [], []
        for _p in range(4):
            mv = jnp.full((16,), inf, jnp.float32)
            mi = jnp.full((16,), _BIG_I, jnp.int32)
            for j in range(16):
                v = lv[pl.ds(j * 16, 16)]
                ix = li[pl.ds(j * 16, 16)]
                take = (v < mv) | ((v == mv) & (ix < mi))
                mv = jnp.where(take, v, mv)
                mi = jnp.where(take, ix, mi)
            minval = mv[0]
            minidx = mi[0]
            for l in range(1, 16):
                v = mv[l]
                ix = mi[l]
                take = (v < minval) | ((v == minval) & (ix < minidx))
                minval = jnp.where(take, v, minval)
                minidx = jnp.where(take, ix, minidx)
            cand_v.append(minval)
            cand_i.append(minidx)
            for j in range(16):
                v = lv[pl.ds(j * 16, 16)]
                ix = li[pl.ds(j * 16, 16)]
                lv[pl.ds(j * 16, 16)] = jnp.where(
                    (v == minval) & (ix == minidx), inf, v)

        symv = sym_v[...]
        candv = jnp.zeros((16,), jnp.int32)
        valv = jnp.full((16,), inf, jnp.float32)
        for p in range(4):
            candv = jnp.where(lanes == p + 1, cand_i[p], candv)
            valv = jnp.where(lanes == p + 1, cand_v[p], valv)

        gidx[...] = symv * _K + candv
        pltpu.async_copy(adj_hbm.at[gidx], ga, sem).wait()
        gidx[...] = symv
        pltpu.async_copy(dists_hbm.at[gidx], gd, sem).wait()

        gdv = gd[...]
        d_sym = gdv[0]
        scorev = jnp.where(lanes == 0, d_sym,
                           valv + jnp.float32(0.1) * ga[...])
        idv = jnp.where(lanes == 0, symv, candv)
        best_s = scorev[0]
        best_i = idv[0]
        for l in range(1, 5):
            s = scorev[l]
            take = s < best_s
            best_s = jnp.where(take, s, best_s)
            best_i = jnp.where(take, idv[l], best_i)
        ob[...] = jnp.zeros((16,), jnp.int32) + best_i
        osc[...] = jnp.zeros((16,), jnp.float32) + best_s
        pltpu.sync_copy(ob, best_hbm)
        pltpu.sync_copy(osc, score_hbm)


def _sc_select(dists, adj_flat, sym16):
    mesh = plsc.VectorSubcoreMesh(
        core_axis_name="c", subcore_axis_name="s",
        num_cores=1, num_subcores=_NTILE)
    f = pl.kernel(
        _sc_body,
        out_type=(
            jax.ShapeDtypeStruct((16,), jnp.int32),
            jax.ShapeDtypeStruct((16,), jnp.float32),
        ),
        mesh=mesh,
        scratch_types=[
            pltpu.VMEM((_CHUNK,), jnp.float32),
            pltpu.VMEM((256,), jnp.float32),
            pltpu.VMEM((256,), jnp.int32),
            pltpu.VMEM((16,), jnp.float32),
            pltpu.VMEM((16,), jnp.int32),
            pltpu.VMEM((16,), jnp.int32),
            pltpu.VMEM((16,), jnp.int32),
            pltpu.VMEM((16,), jnp.float32),
            pltpu.VMEM((16,), jnp.float32),
            pltpu.VMEM((16,), jnp.int32),
            pltpu.VMEM((16,), jnp.float32),
            pltpu.VMEM_SHARED((256,), jnp.float32),
            pltpu.VMEM_SHARED((256,), jnp.int32),
            pltpu.SemaphoreType.DMA,
        ],
    )
    return f(dists, adj_flat, sym16)


def kernel(z_flat, codebook, adjacency, current_sym):
    dists = _tc_dists(z_flat, codebook)
    sym16 = jnp.full((16,), current_sym, dtype=jnp.int32)
    adj_flat = adjacency.reshape(-1)
    best16, score16 = _sc_select(dists, adj_flat, sym16)
    return best16[0], score16[0]


# speedup vs baseline: 6.7821x; 6.7821x over previous
import jax
import jax.numpy as jnp
import numpy as np
from jax import lax
from jax.experimental import pallas as pl
from jax.experimental.pallas import tpu as pltpu
from jax.experimental.pallas import tpu_sc as plsc

_K = 8192
_D = 256
_NTILE = 16
_CHUNK = _K // _NTILE
_BIG_I = np.int32(2**30)


def _dists_body(z_ref, cb_ref, o_ref):
    d = cb_ref[...] - z_ref[...]
    o_ref[...] = jnp.sum(d * d, axis=1)


def _tc_dists(z_flat, codebook):
    return pl.pallas_call(
        _dists_body,
        grid=(8,),
        in_specs=[
            pl.BlockSpec((1, _D), lambda i: (0, 0)),
            pl.BlockSpec((1024, _D), lambda i: (i, 0)),
        ],
        out_specs=pl.BlockSpec((1024,), lambda i: (i,)),
        out_shape=jax.ShapeDtypeStruct((_K,), jnp.float32),
    )(z_flat.reshape(1, _D), codebook)


def _sc_body(dists_hbm, adj_hbm, sym_hbm, best_hbm, score_hbm,
             dv, lv, li, stage_f, stage_i, sym_v, ga, gd,
             ob, osc, sh_v, sh_i, sem):
    t = lax.axis_index("s")
    lanes = lax.iota(jnp.int32, 16)
    inf = jnp.float32(jnp.inf)

    pltpu.sync_copy(dists_hbm.at[pl.ds(t * _CHUNK, _CHUNK)], dv)

    loc_v, loc_i = [], []
    for _p in range(4):
        mv = jnp.full((16,), inf, jnp.float32)
        ev = jnp.full((16,), _BIG_I, jnp.int32)
        for j in range(_CHUNK // 16):
            v = dv[pl.ds(j * 16, 16)]
            c = v < mv
            mv = jnp.where(c, v, mv)
            ev = jnp.where(c, lanes + j * 16, ev)
        minval = mv[0]
        minpos = ev[0]
        for l in range(1, 16):
            v = mv[l]
            e = ev[l]
            take = (v < minval) | ((v == minval) & (e < minpos))
            minval = jnp.where(take, v, minval)
            minpos = jnp.where(take, e, minpos)
        loc_v.append(minval)
        loc_i.append(t * _CHUNK + minpos)
        start = (minpos // 16) * 16
        blk = dv[pl.ds(start, 16)]
        dv[pl.ds(start, 16)] = jnp.where(lanes + start == minpos, inf, blk)

    v4 = jnp.full((16,), inf, jnp.float32)
    i4 = jnp.zeros((16,), jnp.int32)
    for p in range(4):
        v4 = jnp.where(lanes == p, loc_v[p], v4)
        i4 = jnp.where(lanes == p, loc_i[p], i4)
    stage_f[...] = v4
    stage_i[...] = i4
    pltpu.sync_copy(stage_f, sh_v.at[pl.ds(t * 16, 16)])
    pltpu.sync_copy(stage_i, sh_i.at[pl.ds(t * 16, 16)])
    plsc.subcore_barrier()

    @pl.when(t == 0)
    def _():
        pltpu.sync_copy(sh_v, lv)
        pltpu.sync_copy(sh_i, li)
        pltpu.sync_copy(sym_hbm, sym_v)

        cand_v, cand_i = [], []
        for _p in range(4):
            mv = jnp.full((16,), inf, jnp.float32)
            mi = jnp.full((16,), _BIG_I, jnp.int32)
            for j in range(16):
                v = lv[pl.ds(j * 16, 16)]
                ix = li[pl.ds(j * 16, 16)]
                take = (v < mv) | ((v == mv) & (ix < mi))
                mv = jnp.where(take, v, mv)
                mi = jnp.where(take, ix, mi)
            minval = mv[0]
            minidx = mi[0]
            for l in range(1, 16):
                v = mv[l]
                ix = mi[l]
                take = (v < minval) | ((v == minval) & (ix < minidx))
                minval = jnp.where(take, v, minval)
                minidx = jnp.where(take, ix, minidx)
            cand_v.append(minval)
            cand_i.append(minidx)
            for j in range(16):
                v = lv[pl.ds(j * 16, 16)]
                ix = li[pl.ds(j * 16, 16)]
                lv[pl.ds(j * 16, 16)] = jnp.where(
                    (v == minval) & (ix == minidx), inf, v)

        symv = sym_v[...]
        sym0 = symv[0]

        def pick_lane(vec, lane):
            r = vec[0]
            for l in range(1, 16):
                r = jnp.where(lane == l, vec[l], r)
            return r

        dbase = (sym0 // 16) * 16
        pltpu.sync_copy(dists_hbm.at[pl.ds(dbase, 16)], gd)
        d_sym = pick_lane(gd[...], sym0 - dbase)

        best_s = d_sym
        best_i = sym0
        for p in range(4):
            abase = (cand_i[p] // 16) * 16
            pltpu.sync_copy(adj_hbm.at[sym0, pl.ds(abase, 16)],
                            ga.at[pl.ds(p * 16, 16)])
        for p in range(4):
            a = pick_lane(ga[pl.ds(p * 16, 16)], cand_i[p] - (cand_i[p] // 16) * 16)
            s = cand_v[p] + jnp.float32(0.1) * a
            take = s < best_s
            best_s = jnp.where(take, s, best_s)
            best_i = jnp.where(take, cand_i[p], best_i)
        ob[...] = jnp.zeros((16,), jnp.int32) + best_i
        osc[...] = jnp.zeros((16,), jnp.float32) + best_s
        pltpu.sync_copy(ob, best_hbm)
        pltpu.sync_copy(osc, score_hbm)


def _sc_select(dists, adjacency, sym16):
    mesh = plsc.VectorSubcoreMesh(
        core_axis_name="c", subcore_axis_name="s",
        num_cores=1, num_subcores=_NTILE)
    f = pl.kernel(
        _sc_body,
        out_type=(
            jax.ShapeDtypeStruct((16,), jnp.int32),
            jax.ShapeDtypeStruct((16,), jnp.float32),
        ),
        mesh=mesh,
        scratch_types=[
            pltpu.VMEM((_CHUNK,), jnp.float32),
            pltpu.VMEM((256,), jnp.float32),
            pltpu.VMEM((256,), jnp.int32),
            pltpu.VMEM((16,), jnp.float32),
            pltpu.VMEM((16,), jnp.int32),
            pltpu.VMEM((16,), jnp.int32),
            pltpu.VMEM((64,), jnp.float32),
            pltpu.VMEM((16,), jnp.float32),
            pltpu.VMEM((16,), jnp.int32),
            pltpu.VMEM((16,), jnp.float32),
            pltpu.VMEM_SHARED((256,), jnp.float32),
            pltpu.VMEM_SHARED((256,), jnp.int32),
            pltpu.SemaphoreType.DMA,
        ],
    )
    return f(dists, adjacency, sym16)


def kernel(z_flat, codebook, adjacency, current_sym):
    dists = _tc_dists(z_flat, codebook)
    sym16 = jnp.full((16,), current_sym, dtype=jnp.int32)
    best16, score16 = _sc_select(dists, adjacency, sym16)
    return best16[0], score16[0]


# speedup vs baseline: 6.7950x; 1.0019x over previous
import jax
import jax.numpy as jnp
import numpy as np
from jax import lax
from jax.experimental import pallas as pl
from jax.experimental.pallas import tpu as pltpu
from jax.experimental.pallas import tpu_sc as plsc

_K = 8192
_D = 256
_NTILE = 16
_CHUNK = _K // _NTILE
_BIG_I = np.int32(2**30)


def _dists_body(z_ref, cb_ref, o_ref):
    d = cb_ref[...] - z_ref[...]
    ones = jnp.ones((1, _D), jnp.float32)
    res = lax.dot_general(
        ones, d * d, (((1,), (1,)), ((), ())),
        precision=lax.Precision.HIGHEST,
        preferred_element_type=jnp.float32)
    o_ref[...] = res[0]


def _tc_dists(z_flat, codebook):
    return pl.pallas_call(
        _dists_body,
        grid=(8,),
        in_specs=[
            pl.BlockSpec((1, _D), lambda i: (0, 0)),
            pl.BlockSpec((1024, _D), lambda i: (i, 0)),
        ],
        out_specs=pl.BlockSpec((1024,), lambda i: (i,)),
        out_shape=jax.ShapeDtypeStruct((_K,), jnp.float32),
    )(z_flat.reshape(1, _D), codebook)


def _lex_bcast(vv, ii, rrf, rri):
    for s in (8, 4, 2, 1):
        rrf[pl.ds(0, 16)] = vv
        rrf[pl.ds(16, 16)] = vv
        rri[pl.ds(0, 16)] = ii
        rri[pl.ds(16, 16)] = ii
        v2 = rrf[pl.ds(s, 16)]
        i2 = rri[pl.ds(s, 16)]
        take = (v2 < vv) | ((v2 == vv) & (i2 < ii))
        vv = jnp.where(take, v2, vv)
        ii = jnp.where(take, i2, ii)
    return vv, ii


def _min_bcast_f(vv, rrf):
    for s in (8, 4, 2, 1):
        rrf[pl.ds(0, 16)] = vv
        rrf[pl.ds(16, 16)] = vv
        vv = jnp.minimum(vv, rrf[pl.ds(s, 16)])
    return vv


def _min_bcast_i(ii, rri):
    for s in (8, 4, 2, 1):
        rri[pl.ds(0, 16)] = ii
        rri[pl.ds(16, 16)] = ii
        ii = jnp.minimum(ii, rri[pl.ds(s, 16)])
    return ii


def _sc_body(dists_hbm, adj_hbm, sym_hbm, best_hbm, score_hbm,
             dv, sym_v, ga, gd, ob, osc, rrf, rri, lv, li,
             sh_v, sh_i, sem):
    t = lax.axis_index("s")
    lanes = lax.iota(jnp.int32, 16)
    inf = jnp.float32(jnp.inf)

    pltpu.sync_copy(dists_hbm.at[pl.ds(t * _CHUNK, _CHUNK)], dv)

    v4 = jnp.full((16,), inf, jnp.float32)
    i4 = jnp.zeros((16,), jnp.int32)
    for p in range(4):
        mv = jnp.full((16,), inf, jnp.float32)
        ev = jnp.full((16,), _BIG_I, jnp.int32)
        for j in range(_CHUNK // 16):
            v = dv[pl.ds(j * 16, 16)]
            c = v < mv
            mv = jnp.where(c, v, mv)
            ev = jnp.where(c, lanes + j * 16, ev)
        bv, be = _lex_bcast(mv, ev, rrf, rri)
        v4 = jnp.where(lanes == p, bv, v4)
        i4 = jnp.where(lanes == p, be + t * _CHUNK, i4)
        if p < 3:
            minpos = be[0]
            start = (minpos // 16) * 16
            blk = dv[pl.ds(start, 16)]
            dv[pl.ds(start, 16)] = jnp.where(lanes + start == minpos, inf, blk)

    osc[...] = v4
    ob[...] = i4
    pltpu.sync_copy(osc, sh_v.at[pl.ds(t * 16, 16)])
    pltpu.sync_copy(ob, sh_i.at[pl.ds(t * 16, 16)])
    plsc.subcore_barrier()

    @pl.when(t == 0)
    def _():
        pltpu.sync_copy(sh_v, lv)
        pltpu.sync_copy(sh_i, li)
        pltpu.sync_copy(sym_hbm, sym_v)
        symv = sym_v[...]
        sym0 = symv[0]

        dbase = (sym0 // 16) * 16
        cp_d = pltpu.async_copy(dists_hbm.at[pl.ds(dbase, 16)], gd, sem)

        cvals, cidxv, cps, alanes = [], [], [], []
        for p in range(4):
            mv = jnp.full((16,), inf, jnp.float32)
            mi = jnp.full((16,), _BIG_I, jnp.int32)
            for j in range(16):
                v = lv[pl.ds(j * 16, 16)]
                ix = li[pl.ds(j * 16, 16)]
                take = (v < mv) | ((v == mv) & (ix < mi))
                mv = jnp.where(take, v, mv)
                mi = jnp.where(take, ix, mi)
            bv, bidx = _lex_bcast(mv, mi, rrf, rri)
            cvals.append(bv)
            cidxv.append(bidx)
            ci = bidx[0]
            abase = (ci // 16) * 16
            alanes.append(ci - abase)
            cps.append(pltpu.async_copy(
                adj_hbm.at[sym0, pl.ds(abase, 16)],
                ga.at[pl.ds(p * 16, 16)], sem))
            if p < 3:
                for j in range(16):
                    v = lv[pl.ds(j * 16, 16)]
                    ix = li[pl.ds(j * 16, 16)]
                    lv[pl.ds(j * 16, 16)] = jnp.where(
                        (v == bv) & (ix == bidx), inf, v)

        cp_d.wait()
        for cp in cps:
            cp.wait()

        dsel = jnp.where(lanes == sym0 - dbase, gd[...], inf)
        d_sym = _min_bcast_f(dsel, rrf)

        scorev = jnp.where(lanes == 0, d_sym, inf)
        idv = jnp.where(lanes == 0, symv, 0)
        for p in range(4):
            gav = ga[pl.ds(p * 16, 16)]
            asel = jnp.where(lanes == alanes[p], gav, inf)
            a_p = _min_bcast_f(asel, rrf)
            sc = cvals[p] + jnp.float32(0.1) * a_p
            scorev = jnp.where(lanes == p + 1, sc, scorev)
            idv = jnp.where(lanes == p + 1, cidxv[p], idv)

        bs, bl = _lex_bcast(scorev, lanes, rrf, rri)
        bi = _min_bcast_i(jnp.where(lanes == bl, idv, _BIG_I), rri)
        ob[...] = bi
        osc[...] = bs
        pltpu.sync_copy(ob, best_hbm)
        pltpu.sync_copy(osc, score_hbm)


def _sc_select(dists, adjacency, sym16):
    mesh = plsc.VectorSubcoreMesh(
        core_axis_name="c", subcore_axis_name="s",
        num_cores=1, num_subcores=_NTILE)
    f = pl.kernel(
        _sc_body,
        out_type=(
            jax.ShapeDtypeStruct((16,), jnp.int32),
            jax.ShapeDtypeStruct((16,), jnp.float32),
        ),
        mesh=mesh,
        scratch_types=[
            pltpu.VMEM((_CHUNK,), jnp.float32),
            pltpu.VMEM((16,), jnp.int32),
            pltpu.VMEM((64,), jnp.float32),
            pltpu.VMEM((16,), jnp.float32),
            pltpu.VMEM((16,), jnp.int32),
            pltpu.VMEM((16,), jnp.float32),
            pltpu.VMEM((32,), jnp.float32),
            pltpu.VMEM((32,), jnp.int32),
            pltpu.VMEM((256,), jnp.float32),
            pltpu.VMEM((256,), jnp.int32),
            pltpu.VMEM_SHARED((256,), jnp.float32),
            pltpu.VMEM_SHARED((256,), jnp.int32),
            pltpu.SemaphoreType.DMA,
        ],
    )
    return f(dists, adjacency, sym16)


def kernel(z_flat, codebook, adjacency, current_sym):
    dists = _tc_dists(z_flat, codebook)
    sym16 = jnp.full((16,), current_sym, dtype=jnp.int32)
    best16, score16 = _sc_select(dists, adjacency, sym16)
    return best16[0], score16[0]


# speedup vs baseline: 7.7038x; 1.1337x over previous
import jax
import jax.numpy as jnp
import numpy as np
from jax import lax
from jax.experimental import pallas as pl
from jax.experimental.pallas import tpu as pltpu
from jax.experimental.pallas import tpu_sc as plsc

_K = 8192
_D = 256
_NTILE = 16
_CHUNK = _K // _NTILE
_BIG_I = np.int32(2**30)


_TC_BLK = 2048
_TC_GRID = _K // _TC_BLK


def _dists_body(z_ref, cb_ref, o_ref):
    d = cb_ref[...] - z_ref[...]
    o_ref[...] = jnp.sum(d * d, axis=1)


def _tc_dists(z_flat, codebook):
    return pl.pallas_call(
        _dists_body,
        grid=(_TC_GRID,),
        in_specs=[
            pl.BlockSpec((1, _D), lambda i: (0, 0)),
            pl.BlockSpec((_TC_BLK, _D), lambda i: (i, 0)),
        ],
        out_specs=pl.BlockSpec((_TC_BLK,), lambda i: (i,)),
        out_shape=jax.ShapeDtypeStruct((_K,), jnp.float32),
    )(z_flat.reshape(1, _D), codebook)


def _lex_bcast(vv, ii, rrf, rri):
    for s in (8, 4, 2, 1):
        rrf[pl.ds(0, 16)] = vv
        rrf[pl.ds(16, 16)] = vv
        rri[pl.ds(0, 16)] = ii
        rri[pl.ds(16, 16)] = ii
        v2 = rrf[pl.ds(s, 16)]
        i2 = rri[pl.ds(s, 16)]
        take = (v2 < vv) | ((v2 == vv) & (i2 < ii))
        vv = jnp.where(take, v2, vv)
        ii = jnp.where(take, i2, ii)
    return vv, ii


def _min_bcast_f(vv, rrf):
    for s in (8, 4, 2, 1):
        rrf[pl.ds(0, 16)] = vv
        rrf[pl.ds(16, 16)] = vv
        vv = jnp.minimum(vv, rrf[pl.ds(s, 16)])
    return vv


def _min_bcast_i(ii, rri):
    for s in (8, 4, 2, 1):
        rri[pl.ds(0, 16)] = ii
        rri[pl.ds(16, 16)] = ii
        ii = jnp.minimum(ii, rri[pl.ds(s, 16)])
    return ii


def _sc_body(dists_hbm, adj_hbm, sym_hbm, best_hbm, score_hbm,
             dv, sym_v, ga, gd, ob, osc, rrf, rri, lv, li,
             sh_v, sh_i, sem):
    t = lax.axis_index("s")
    lanes = lax.iota(jnp.int32, 16)
    inf = jnp.float32(jnp.inf)

    pltpu.sync_copy(dists_hbm.at[pl.ds(t * _CHUNK, _CHUNK)], dv)

    v4 = jnp.full((16,), inf, jnp.float32)
    i4 = jnp.zeros((16,), jnp.int32)
    for p in range(4):
        mv = jnp.full((16,), inf, jnp.float32)
        ev = jnp.full((16,), _BIG_I, jnp.int32)
        for j in range(_CHUNK // 16):
            v = dv[pl.ds(j * 16, 16)]
            c = v < mv
            mv = jnp.where(c, v, mv)
            ev = jnp.where(c, lanes + j * 16, ev)
        bv, be = _lex_bcast(mv, ev, rrf, rri)
        v4 = jnp.where(lanes == p, bv, v4)
        i4 = jnp.where(lanes == p, be + t * _CHUNK, i4)
        if p < 3:
            minpos = be[0]
            start = (minpos // 16) * 16
            blk = dv[pl.ds(start, 16)]
            dv[pl.ds(start, 16)] = jnp.where(lanes + start == minpos, inf, blk)

    osc[...] = v4
    ob[...] = i4
    pltpu.sync_copy(osc, sh_v.at[pl.ds(t * 16, 16)])
    pltpu.sync_copy(ob, sh_i.at[pl.ds(t * 16, 16)])
    plsc.subcore_barrier()

    @pl.when(t == 0)
    def _():
        pltpu.sync_copy(sh_v, lv)
        pltpu.sync_copy(sh_i, li)
        pltpu.sync_copy(sym_hbm, sym_v)
        symv = sym_v[...]
        sym0 = symv[0]

        dbase = (sym0 // 16) * 16
        cp_d = pltpu.async_copy(dists_hbm.at[pl.ds(dbase, 16)], gd, sem)

        cvals, cidxv, cps, alanes = [], [], [], []
        for p in range(4):
            mv = jnp.full((16,), inf, jnp.float32)
            mi = jnp.full((16,), _BIG_I, jnp.int32)
            for j in range(16):
                v = lv[pl.ds(j * 16, 16)]
                ix = li[pl.ds(j * 16, 16)]
                take = (v < mv) | ((v == mv) & (ix < mi))
                mv = jnp.where(take, v, mv)
                mi = jnp.where(take, ix, mi)
            bv, bidx = _lex_bcast(mv, mi, rrf, rri)
            cvals.append(bv)
            cidxv.append(bidx)
            ci = bidx[0]
            abase = (ci // 16) * 16
            alanes.append(ci - abase)
            cps.append(pltpu.async_copy(
                adj_hbm.at[sym0, pl.ds(abase, 16)],
                ga.at[pl.ds(p * 16, 16)], sem))
            if p < 3:
                for j in range(16):
                    v = lv[pl.ds(j * 16, 16)]
                    ix = li[pl.ds(j * 16, 16)]
                    lv[pl.ds(j * 16, 16)] = jnp.where(
                        (v == bv) & (ix == bidx), inf, v)

        cp_d.wait()
        for cp in cps:
            cp.wait()

        dsel = jnp.where(lanes == sym0 - dbase, gd[...], inf)
        d_sym = _min_bcast_f(dsel, rrf)

        scorev = jnp.where(lanes == 0, d_sym, inf)
        idv = jnp.where(lanes == 0, symv, 0)
        for p in range(4):
            gav = ga[pl.ds(p * 16, 16)]
            asel = jnp.where(lanes == alanes[p], gav, inf)
            a_p = _min_bcast_f(asel, rrf)
            sc = cvals[p] + jnp.float32(0.1) * a_p
            scorev = jnp.where(lanes == p + 1, sc, scorev)
            idv = jnp.where(lanes == p + 1, cidxv[p], idv)

        bs, bl = _lex_bcast(scorev, lanes, rrf, rri)
        bi = _min_bcast_i(jnp.where(lanes == bl, idv, _BIG_I), rri)
        ob[...] = bi
        osc[...] = bs
        pltpu.sync_copy(ob, best_hbm)
        pltpu.sync_copy(osc, score_hbm)


def _sc_select(dists, adjacency, sym16):
    mesh = plsc.VectorSubcoreMesh(
        core_axis_name="c", subcore_axis_name="s",
        num_cores=1, num_subcores=_NTILE)
    f = pl.kernel(
        _sc_body,
        out_type=(
            jax.ShapeDtypeStruct((16,), jnp.int32),
            jax.ShapeDtypeStruct((16,), jnp.float32),
        ),
        mesh=mesh,
        scratch_types=[
            pltpu.VMEM((_CHUNK,), jnp.float32),
            pltpu.VMEM((16,), jnp.int32),
            pltpu.VMEM((64,), jnp.float32),
            pltpu.VMEM((16,), jnp.float32),
            pltpu.VMEM((16,), jnp.int32),
            pltpu.VMEM((16,), jnp.float32),
            pltpu.VMEM((32,), jnp.float32),
            pltpu.VMEM((32,), jnp.int32),
            pltpu.VMEM((256,), jnp.float32),
            pltpu.VMEM((256,), jnp.int32),
            pltpu.VMEM_SHARED((256,), jnp.float32),
            pltpu.VMEM_SHARED((256,), jnp.int32),
            pltpu.SemaphoreType.DMA,
        ],
    )
    return f(dists, adjacency, sym16)


def kernel(z_flat, codebook, adjacency, current_sym):
    dists = _tc_dists(z_flat, codebook)
    sym16 = jnp.full((16,), current_sym, dtype=jnp.int32)
    best16, score16 = _sc_select(dists, adjacency, sym16)
    return best16[0], score16[0]
